# baseline (speedup 1.0000x reference)
import functools

import jax
import jax.numpy as jnp
from jax import lax
from jax.experimental import pallas as pl
from jax.experimental.pallas import tpu as pltpu
from jax.experimental.pallas import tpu_sc as plsc

F32 = jnp.float32
CHUNK = 128
NW = 32


def _proj_body(nf_ref, ws_ref, wr_ref, ps_ref, pr_ref):
    nf = nf_ref[...]
    ps_ref[...] = jnp.dot(nf, ws_ref[...], preferred_element_type=F32)
    pr_ref[...] = jnp.dot(nf, wr_ref[...], preferred_element_type=F32)


def _edge_body(ef_ref, gs_ref, we_ref, be1_ref, we2_ref, be2_ref,
               sc_ref, bi_ref, ue_ref, ne_ref):
    ef = ef_ref[...]
    x = (gs_ref[...] + be1_ref[...]
         + jnp.dot(ef, we_ref[...], preferred_element_type=F32))
    h = jnp.maximum(x, 0.0)
    y = jnp.dot(h, we2_ref[...], preferred_element_type=F32) + be2_ref[...]
    mean = jnp.mean(y, axis=-1, keepdims=True)
    var = jnp.mean((y - mean) ** 2, axis=-1, keepdims=True)
    ue = (y - mean) / jnp.sqrt(var + 1e-5) * sc_ref[...] + bi_ref[...]
    ue_ref[...] = ue
    ne_ref[...] = ue + ef


def _node_body(nf_ref, p0_ref, p1_ref, wa_ref, wb_ref, bn1_ref, wn2_ref,
               bn2_ref, sc_ref, bi_ref, out_ref):
    nf = nf_ref[...]
    agg = p0_ref[...] + p1_ref[...]
    x = (jnp.dot(nf, wa_ref[...], preferred_element_type=F32)
         + jnp.dot(agg, wb_ref[...], preferred_element_type=F32)
         + bn1_ref[...])
    h = jnp.maximum(x, 0.0)
    y = jnp.dot(h, wn2_ref[...], preferred_element_type=F32) + bn2_ref[...]
    mean = jnp.mean(y, axis=-1, keepdims=True)
    var = jnp.mean((y - mean) ** 2, axis=-1, keepdims=True)
    out_ref[...] = ((y - mean) / jnp.sqrt(var + 1e-5) * sc_ref[...]
                    + bi_ref[...] + nf)


def _tc_proj(nf, ws, wr):
    n, d = nf.shape
    bn = 2000
    return pl.pallas_call(
        _proj_body,
        grid=(n // bn,),
        in_specs=[
            pl.BlockSpec((bn, d), lambda i: (i, 0)),
            pl.BlockSpec((d, d), lambda i: (0, 0)),
            pl.BlockSpec((d, d), lambda i: (0, 0)),
        ],
        out_specs=(pl.BlockSpec((bn, d), lambda i: (i, 0)),
                   pl.BlockSpec((bn, d), lambda i: (i, 0))),
        out_shape=(jax.ShapeDtypeStruct((n, d), F32),
                   jax.ShapeDtypeStruct((n, d), F32)),
    )(nf, ws, wr)


def _tc_edge(ef, gs, we, be1, we2, be2, sc, bi):
    e, d = ef.shape
    be = 2000
    row = lambda i: (i, 0)
    cst = lambda i: (0, 0)
    return pl.pallas_call(
        _edge_body,
        grid=(e // be,),
        in_specs=[
            pl.BlockSpec((be, d), row),
            pl.BlockSpec((be, d), row),
            pl.BlockSpec((d, d), cst),
            pl.BlockSpec((1, d), cst),
            pl.BlockSpec((d, d), cst),
            pl.BlockSpec((1, d), cst),
            pl.BlockSpec((1, d), cst),
            pl.BlockSpec((1, d), cst),
        ],
        out_specs=(pl.BlockSpec((be, d), row), pl.BlockSpec((be, d), row)),
        out_shape=(jax.ShapeDtypeStruct((e, d), F32),
                   jax.ShapeDtypeStruct((e, d), F32)),
    )(ef, gs, we, be1, we2, be2, sc, bi)


def _tc_node(nf, p0, p1, wa, wb, bn1, wn2, bn2, sc, bi):
    n, d = nf.shape
    bn = 2000
    row = lambda i: (i, 0)
    cst = lambda i: (0, 0)
    return pl.pallas_call(
        _node_body,
        grid=(n // bn,),
        in_specs=[
            pl.BlockSpec((bn, d), row),
            pl.BlockSpec((bn, d), row),
            pl.BlockSpec((bn, d), row),
            pl.BlockSpec((d, d), cst),
            pl.BlockSpec((d, d), cst),
            pl.BlockSpec((1, d), cst),
            pl.BlockSpec((d, d), cst),
            pl.BlockSpec((1, d), cst),
            pl.BlockSpec((1, d), cst),
            pl.BlockSpec((1, d), cst),
        ],
        out_specs=pl.BlockSpec((bn, d), row),
        out_shape=jax.ShapeDtypeStruct((n, d), F32),
    )(nf, p0, p1, wa, wb, bn1, wn2, bn2, sc, bi)


@functools.lru_cache(maxsize=None)
def _make_gather(n_chunks, d):
    mesh = plsc.VectorSubcoreMesh(core_axis_name="c", subcore_axis_name="s")
    e = n_chunks * CHUNK
    n_iter = n_chunks // NW

    @functools.partial(
        pl.kernel,
        out_type=jax.ShapeDtypeStruct((e, d), F32),
        mesh=mesh,
        scratch_types=[
            pltpu.VMEM((CHUNK,), jnp.int32),
            pltpu.VMEM((CHUNK,), jnp.int32),
            pltpu.VMEM((CHUNK,), jnp.int32),
            pltpu.VMEM((CHUNK,), jnp.int32),
            pltpu.VMEM((CHUNK, d), F32),
            pltpu.VMEM((CHUNK, d), F32),
            pltpu.VMEM((CHUNK, d), F32),
            pltpu.VMEM((CHUNK, d), F32),
            pltpu.VMEM((CHUNK,), jnp.int32),
            pltpu.SemaphoreType.DMA,
            pltpu.SemaphoreType.DMA,
            pltpu.SemaphoreType.DMA,
            pltpu.SemaphoreType.DMA,
        ],
    )
    def gather_k(ps_hbm, pr_hbm, sidx_hbm, ridx_hbm, out_hbm,
                 si0, si1, ri0, ri1, rs0, rs1, rr0, rr1, ident,
                 ss0, ss1, sr0, sr1):
        wid = lax.axis_index("s") * 2 + lax.axis_index("c")
        B = ((si0, ri0, rs0, rr0, ss0, sr0),
             (si1, ri1, rs1, rr1, ss1, sr1))

        for q in range(CHUNK // 16):
            ident[pl.ds(q * 16, 16)] = lax.iota(jnp.int32, 16) + q * 16

        def issue(j, b):
            si, ri, rs, rr, ssem, rsem = B[b]
            c = j * NW + wid
            pltpu.sync_copy(sidx_hbm.at[pl.ds(c * CHUNK, CHUNK)], si)
            pltpu.sync_copy(ridx_hbm.at[pl.ds(c * CHUNK, CHUNK)], ri)
            pltpu.async_copy(ps_hbm.at[si], rs, ssem)
            pltpu.async_copy(pr_hbm.at[ri], rr, rsem)

        def finish(j, b):
            si, ri, rs, rr, ssem, rsem = B[b]
            c = j * NW + wid
            pltpu.make_async_copy(ps_hbm.at[si], rs, ssem).wait()
            pltpu.make_async_copy(pr_hbm.at[ri], rr, rsem).wait()

            @plsc.parallel_loop(0, CHUNK, step=1, unroll=8)
            def _add(r):
                for q in range(d // 16):
                    sl = pl.ds(q * 16, 16)
                    rs[r, sl] = rs[r, sl] + rr[r, sl]

            pltpu.sync_copy(rs, out_hbm.at[pl.ds(c * CHUNK, CHUNK)])

        issue(0, 0)

        def body(i, carry):
            j0 = 2 * i
            issue(j0 + 1, 1)
            finish(j0, 0)
            issue(j0 + 2, 0)
            finish(j0 + 1, 1)
            return carry

        lax.fori_loop(0, n_iter // 2 - 1, body, 0)
        issue(n_iter - 1, 1)
        finish(n_iter - 2, 0)
        finish(n_iter - 1, 1)

    return gather_k


@functools.lru_cache(maxsize=None)
def _make_scatter(n_nodes, n_chunks, d):
    mesh = plsc.VectorSubcoreMesh(core_axis_name="c", subcore_axis_name="s")
    n_pad = ((n_nodes + 16 * 128 - 1) // (16 * 128)) * 16 * 128
    rows_per_tile = n_pad // 16
    zr = 64
    n_zcopy = rows_per_tile // zr

    n_full = n_chunks // NW
    extra = n_chunks % NW
    assert n_full >= 4 and n_full % 2 == 0

    @functools.partial(
        pl.kernel,
        out_type=jax.ShapeDtypeStruct((2, n_pad, d), F32),
        mesh=mesh,
        scratch_types=[
            pltpu.VMEM((CHUNK,), jnp.int32),
            pltpu.VMEM((CHUNK,), jnp.int32),
            pltpu.VMEM((CHUNK, d), F32),
            pltpu.VMEM((CHUNK, d), F32),
            pltpu.VMEM((zr, d), F32),
            pltpu.VMEM_SHARED((n_pad, d), F32),
            pltpu.SemaphoreType.DMA,
            pltpu.SemaphoreType.DMA,
        ],
    )
    def scatter_k(ue_hbm, ridx_hbm, out_hbm, ib0, ib1, rv0, rv1, zbuf,
                  acc_sh, sm0, sm1):
        cid = lax.axis_index("c")
        sid = lax.axis_index("s")
        wid = sid * 2 + cid
        B = ((ib0, rv0, sm0), (ib1, rv1, sm1))

        def issue(j, b):
            ib, rv, sm = B[b]
            c = j * NW + wid
            pltpu.sync_copy(ridx_hbm.at[pl.ds(c * CHUNK, CHUNK)], ib)
            pltpu.async_copy(ue_hbm.at[pl.ds(c * CHUNK, CHUNK)], rv, sm)

        def finish(j, b):
            ib, rv, sm = B[b]
            c = j * NW + wid
            pltpu.make_async_copy(
                ue_hbm.at[pl.ds(c * CHUNK, CHUNK)], rv, sm).wait()
            pltpu.sync_copy(rv, acc_sh.at[ib], add=True)

        issue(0, 0)

        def zb(i, carry):
            r = i // (d // 16)
            q = (i % (d // 16)) * 16
            zbuf[r, pl.ds(q, 16)] = jnp.zeros((16,), F32)
            return carry

        lax.fori_loop(0, zr * (d // 16), zb, 0)
        base = sid * rows_per_tile
        for t in range(n_zcopy):
            pltpu.sync_copy(zbuf, acc_sh.at[pl.ds(base + t * zr, zr)])
        plsc.subcore_barrier()

        def body(i, carry):
            j0 = 2 * i
            issue(j0 + 1, 1)
            finish(j0, 0)
            issue(j0 + 2, 0)
            finish(j0 + 1, 1)
            return carry

        lax.fori_loop(0, n_full // 2 - 1, body, 0)
        issue(n_full - 1, 1)
        finish(n_full - 2, 0)
        if extra:
            @pl.when(wid < extra)
            def _():
                issue(n_full, 0)
        finish(n_full - 1, 1)
        if extra:
            @pl.when(wid < extra)
            def _():
                finish(n_full, 0)
        plsc.subcore_barrier()
        for t in range(n_zcopy):
            sl = pl.ds(base + t * zr, zr)
            pltpu.sync_copy(acc_sh.at[sl], out_hbm.at[cid, sl])

    return scatter_k


def kernel(node_features, edge_features, senders, receivers,
           We1, be1, We2, be2, ln_e_scale, ln_e_bias,
           Wn1, bn1, Wn2, bn2, ln_n_scale, ln_n_bias):
    n, d = node_features.shape
    e = edge_features.shape[0]
    n_chunks = e // CHUNK

    sidx = senders.astype(jnp.int32)
    ridx = receivers.astype(jnp.int32)

    n_chunks_pad = ((n_chunks + 2 * NW - 1) // (2 * NW)) * 2 * NW
    e_pad = n_chunks_pad * CHUNK
    sidx_p = jnp.pad(sidx, (0, e_pad - e))
    ridx_p = jnp.pad(ridx, (0, e_pad - e))

    ws, wr, we = We1[:d], We1[d:2 * d], We1[2 * d:]
    ps, pr = _tc_proj(node_features, ws, wr)
    gsum = _make_gather(n_chunks_pad, d)(ps, pr, sidx_p, ridx_p)
    ue, ne = _tc_edge(edge_features, gsum, we,
                      be1.reshape(1, d), We2, be2.reshape(1, d),
                      ln_e_scale.reshape(1, d), ln_e_bias.reshape(1, d))
    agg2 = _make_scatter(n, n_chunks, d)(ue, ridx)
    new_nodes = _tc_node(node_features, agg2[0, :n], agg2[1, :n],
                         Wn1[:d], Wn1[d:], bn1.reshape(1, d),
                         Wn2, bn2.reshape(1, d),
                         ln_n_scale.reshape(1, d), ln_n_bias.reshape(1, d))
    return (new_nodes, ne)


# speedup vs baseline: 1.4264x; 1.4264x over previous
import functools

import jax
import jax.numpy as jnp
from jax import lax
from jax.experimental import pallas as pl
from jax.experimental.pallas import tpu as pltpu
from jax.experimental.pallas import tpu_sc as plsc

F32 = jnp.float32
CHUNK = 128
NW = 32


def _proj_body(nf_ref, ws_ref, wr_ref, ps_ref, pr_ref):
    nf = nf_ref[...]
    ps_ref[...] = jnp.dot(nf, ws_ref[...], preferred_element_type=F32)
    pr_ref[...] = jnp.dot(nf, wr_ref[...], preferred_element_type=F32)


def _edge_body(ef_ref, gs_ref, gr_ref, we_ref, be1_ref, we2_ref, be2_ref,
               sc_ref, bi_ref, ue_ref, ne_ref):
    ef = ef_ref[...]
    x = (gs_ref[0] + gr_ref[0] + be1_ref[...]
         + jnp.dot(ef, we_ref[...], preferred_element_type=F32))
    h = jnp.maximum(x, 0.0)
    y = jnp.dot(h, we2_ref[...], preferred_element_type=F32) + be2_ref[...]
    mean = jnp.mean(y, axis=-1, keepdims=True)
    var = jnp.mean((y - mean) ** 2, axis=-1, keepdims=True)
    ue = (y - mean) / jnp.sqrt(var + 1e-5) * sc_ref[...] + bi_ref[...]
    ue_ref[...] = ue
    ne_ref[...] = ue + ef


def _node_body(nf_ref, p0_ref, p1_ref, wa_ref, wb_ref, bn1_ref, wn2_ref,
               bn2_ref, sc_ref, bi_ref, out_ref):
    nf = nf_ref[...]
    agg = p0_ref[...] + p1_ref[...]
    x = (jnp.dot(nf, wa_ref[...], preferred_element_type=F32)
         + jnp.dot(agg, wb_ref[...], preferred_element_type=F32)
         + bn1_ref[...])
    h = jnp.maximum(x, 0.0)
    y = jnp.dot(h, wn2_ref[...], preferred_element_type=F32) + bn2_ref[...]
    mean = jnp.mean(y, axis=-1, keepdims=True)
    var = jnp.mean((y - mean) ** 2, axis=-1, keepdims=True)
    out_ref[...] = ((y - mean) / jnp.sqrt(var + 1e-5) * sc_ref[...]
                    + bi_ref[...] + nf)


def _tc_proj(nf, ws, wr):
    n, d = nf.shape
    bn = 2000
    return pl.pallas_call(
        _proj_body,
        grid=(n // bn,),
        in_specs=[
            pl.BlockSpec((bn, d), lambda i: (i, 0)),
            pl.BlockSpec((d, d), lambda i: (0, 0)),
            pl.BlockSpec((d, d), lambda i: (0, 0)),
        ],
        out_specs=(pl.BlockSpec((bn, d), lambda i: (i, 0)),
                   pl.BlockSpec((bn, d), lambda i: (i, 0))),
        out_shape=(jax.ShapeDtypeStruct((n, d), F32),
                   jax.ShapeDtypeStruct((n, d), F32)),
    )(nf, ws, wr)


def _tc_edge(ef, g2, we, be1, we2, be2, sc, bi):
    e, d = ef.shape
    be = 2000
    row = lambda i: (i, 0)
    cst = lambda i: (0, 0)
    return pl.pallas_call(
        _edge_body,
        grid=(e // be,),
        in_specs=[
            pl.BlockSpec((be, d), row),
            pl.BlockSpec((1, be, d), lambda i: (0, i, 0)),
            pl.BlockSpec((1, be, d), lambda i: (1, i, 0)),
            pl.BlockSpec((d, d), cst),
            pl.BlockSpec((1, d), cst),
            pl.BlockSpec((d, d), cst),
            pl.BlockSpec((1, d), cst),
            pl.BlockSpec((1, d), cst),
            pl.BlockSpec((1, d), cst),
        ],
        out_specs=(pl.BlockSpec((be, d), row), pl.BlockSpec((be, d), row)),
        out_shape=(jax.ShapeDtypeStruct((e, d), F32),
                   jax.ShapeDtypeStruct((e, d), F32)),
    )(ef, g2, g2, we, be1, we2, be2, sc, bi)


def _tc_node(nf, p0, p1, wa, wb, bn1, wn2, bn2, sc, bi):
    n, d = nf.shape
    bn = 2000
    row = lambda i: (i, 0)
    cst = lambda i: (0, 0)
    return pl.pallas_call(
        _node_body,
        grid=(n // bn,),
        in_specs=[
            pl.BlockSpec((bn, d), row),
            pl.BlockSpec((bn, d), row),
            pl.BlockSpec((bn, d), row),
            pl.BlockSpec((d, d), cst),
            pl.BlockSpec((d, d), cst),
            pl.BlockSpec((1, d), cst),
            pl.BlockSpec((d, d), cst),
            pl.BlockSpec((1, d), cst),
            pl.BlockSpec((1, d), cst),
            pl.BlockSpec((1, d), cst),
        ],
        out_specs=pl.BlockSpec((bn, d), row),
        out_shape=jax.ShapeDtypeStruct((n, d), F32),
    )(nf, p0, p1, wa, wb, bn1, wn2, bn2, sc, bi)


@functools.lru_cache(maxsize=None)
def _make_gather(n_nodes, n_chunks, n_chunks_pad, d):
    mesh = plsc.VectorSubcoreMesh(core_axis_name="c", subcore_axis_name="s")
    e_pad = n_chunks_pad * CHUNK
    NS = 16
    n_full = n_chunks // NS
    extra = n_chunks % NS
    assert n_full >= 4 and n_full % 2 == 0
    stripe = ((n_nodes // NS) // 8 + 1) * 8
    last = n_nodes - 15 * stripe
    assert 0 < last <= stripe

    @functools.partial(
        pl.kernel,
        out_type=jax.ShapeDtypeStruct((2, e_pad, d), F32),
        mesh=mesh,
        scratch_types=[
            pltpu.VMEM((CHUNK,), jnp.int32),
            pltpu.VMEM((CHUNK,), jnp.int32),
            pltpu.VMEM((CHUNK, d), F32),
            pltpu.VMEM((CHUNK, d), F32),
            pltpu.VMEM_SHARED((n_nodes, d), F32),
            pltpu.SemaphoreType.DMA,
            pltpu.SemaphoreType.DMA,
        ],
    )
    def gather_k(ps_hbm, pr_hbm, sidx_hbm, ridx_hbm, out_hbm,
                 ib0, ib1, rv0, rv1, tbl_sh, ws0, ws1):
        cid = lax.axis_index("c")
        sid = lax.axis_index("s")
        B = ((ib0, rv0, ws0), (ib1, rv1, ws1))

        @pl.when(sid < 15)
        def _():
            sl = pl.ds(sid * stripe, stripe)

            @pl.when(cid == 0)
            def _():
                pltpu.sync_copy(ps_hbm.at[sl], tbl_sh.at[sl])

            @pl.when(cid == 1)
            def _():
                pltpu.sync_copy(pr_hbm.at[sl], tbl_sh.at[sl])

        @pl.when(sid == 15)
        def _():
            sl = pl.ds(15 * stripe, last)

            @pl.when(cid == 0)
            def _():
                pltpu.sync_copy(ps_hbm.at[sl], tbl_sh.at[sl])

            @pl.when(cid == 1)
            def _():
                pltpu.sync_copy(pr_hbm.at[sl], tbl_sh.at[sl])

        plsc.subcore_barrier()

        def load_idx(j, b):
            ib = B[b][0]
            c = j * NS + sid

            @pl.when(cid == 0)
            def _():
                pltpu.sync_copy(sidx_hbm.at[pl.ds(c * CHUNK, CHUNK)], ib)

            @pl.when(cid == 1)
            def _():
                pltpu.sync_copy(ridx_hbm.at[pl.ds(c * CHUNK, CHUNK)], ib)

        def stage(j, b, first):
            ib, rv, wsem = B[b]
            c = j * NS + sid
            if not first:
                pltpu.make_async_copy(
                    rv, out_hbm.at[cid, pl.ds(0, CHUNK)], wsem).wait()
            load_idx(j + 1, 1 - b)
            pltpu.sync_copy(tbl_sh.at[ib], rv)
            pltpu.async_copy(rv, out_hbm.at[cid, pl.ds(c * CHUNK, CHUNK)],
                             wsem)

        load_idx(0, 0)
        stage(0, 0, True)
        stage(1, 1, True)

        def body(i, carry):
            stage(2 * i, 0, False)
            stage(2 * i + 1, 1, False)
            return carry

        lax.fori_loop(1, n_full // 2, body, 0)
        if extra:
            @pl.when(sid < extra)
            def _():
                stage(n_full, 0, False)
        pltpu.make_async_copy(rv0, out_hbm.at[cid, pl.ds(0, CHUNK)],
                              ws0).wait()
        pltpu.make_async_copy(rv1, out_hbm.at[cid, pl.ds(0, CHUNK)],
                              ws1).wait()

    return gather_k


@functools.lru_cache(maxsize=None)
def _make_scatter(n_nodes, n_chunks, d):
    mesh = plsc.VectorSubcoreMesh(core_axis_name="c", subcore_axis_name="s")
    n_pad = ((n_nodes + 16 * 128 - 1) // (16 * 128)) * 16 * 128
    rows_per_tile = n_pad // 16
    zr = 64
    n_zcopy = rows_per_tile // zr

    n_full = n_chunks // NW
    extra = n_chunks % NW
    assert n_full >= 4 and n_full % 2 == 0

    @functools.partial(
        pl.kernel,
        out_type=jax.ShapeDtypeStruct((2, n_pad, d), F32),
        mesh=mesh,
        scratch_types=[
            pltpu.VMEM((CHUNK,), jnp.int32),
            pltpu.VMEM((CHUNK,), jnp.int32),
            pltpu.VMEM((CHUNK, d), F32),
            pltpu.VMEM((CHUNK, d), F32),
            pltpu.VMEM((zr, d), F32),
            pltpu.VMEM_SHARED((n_pad, d), F32),
            pltpu.SemaphoreType.DMA,
            pltpu.SemaphoreType.DMA,
        ],
    )
    def scatter_k(ue_hbm, ridx_hbm, out_hbm, ib0, ib1, rv0, rv1, zbuf,
                  acc_sh, sm0, sm1):
        cid = lax.axis_index("c")
        sid = lax.axis_index("s")
        wid = sid * 2 + cid
        B = ((ib0, rv0, sm0), (ib1, rv1, sm1))

        def issue(j, b):
            ib, rv, sm = B[b]
            c = j * NW + wid
            pltpu.sync_copy(ridx_hbm.at[pl.ds(c * CHUNK, CHUNK)], ib)
            pltpu.async_copy(ue_hbm.at[pl.ds(c * CHUNK, CHUNK)], rv, sm)

        def finish(j, b):
            ib, rv, sm = B[b]
            c = j * NW + wid
            pltpu.make_async_copy(
                ue_hbm.at[pl.ds(c * CHUNK, CHUNK)], rv, sm).wait()
            pltpu.sync_copy(rv, acc_sh.at[ib], add=True)

        issue(0, 0)

        def zb(i, carry):
            r = i // (d // 16)
            q = (i % (d // 16)) * 16
            zbuf[r, pl.ds(q, 16)] = jnp.zeros((16,), F32)
            return carry

        lax.fori_loop(0, zr * (d // 16), zb, 0)
        base = sid * rows_per_tile
        for t in range(n_zcopy):
            pltpu.sync_copy(zbuf, acc_sh.at[pl.ds(base + t * zr, zr)])
        plsc.subcore_barrier()

        def body(i, carry):
            j0 = 2 * i
            issue(j0 + 1, 1)
            finish(j0, 0)
            issue(j0 + 2, 0)
            finish(j0 + 1, 1)
            return carry

        lax.fori_loop(0, n_full // 2 - 1, body, 0)
        issue(n_full - 1, 1)
        finish(n_full - 2, 0)
        if extra:
            @pl.when(wid < extra)
            def _():
                issue(n_full, 0)
        finish(n_full - 1, 1)
        if extra:
            @pl.when(wid < extra)
            def _():
                finish(n_full, 0)
        plsc.subcore_barrier()
        for t in range(n_zcopy):
            sl = pl.ds(base + t * zr, zr)
            pltpu.sync_copy(acc_sh.at[sl], out_hbm.at[cid, sl])

    return scatter_k


def kernel(node_features, edge_features, senders, receivers,
           We1, be1, We2, be2, ln_e_scale, ln_e_bias,
           Wn1, bn1, Wn2, bn2, ln_n_scale, ln_n_bias):
    n, d = node_features.shape
    e = edge_features.shape[0]
    n_chunks = e // CHUNK

    sidx = senders.astype(jnp.int32)
    ridx = receivers.astype(jnp.int32)

    n_chunks_pad = ((n_chunks + 2 * NW - 1) // (2 * NW)) * 2 * NW
    e_pad = n_chunks_pad * CHUNK
    sidx_p = jnp.pad(sidx, (0, e_pad - e))
    ridx_p = jnp.pad(ridx, (0, e_pad - e))

    ws, wr, we = We1[:d], We1[d:2 * d], We1[2 * d:]
    ps, pr = _tc_proj(node_features, ws, wr)
    g2 = _make_gather(n, n_chunks, n_chunks_pad, d)(ps, pr, sidx_p, ridx_p)
    ue, ne = _tc_edge(edge_features, g2, we,
                      be1.reshape(1, d), We2, be2.reshape(1, d),
                      ln_e_scale.reshape(1, d), ln_e_bias.reshape(1, d))
    agg2 = _make_scatter(n, n_chunks, d)(ue, ridx)
    new_nodes = _tc_node(node_features, agg2[0, :n], agg2[1, :n],
                         Wn1[:d], Wn1[d:], bn1.reshape(1, d),
                         Wn2, bn2.reshape(1, d),
                         ln_n_scale.reshape(1, d), ln_n_bias.reshape(1, d))
    return (new_nodes, ne)
